# baseline (speedup 1.0000x reference)
import functools

import jax
import jax.numpy as jnp
from jax import lax
from jax.experimental import pallas as pl
from jax.experimental.pallas import tpu as pltpu
from jax.experimental.pallas import tpu_sc as plsc


def _make_sc_gather(B, V, D, nc, ns):
    NW = nc * ns
    b_per_w = B // NW
    C = 512
    nchunks = b_per_w // C
    mesh = plsc.VectorSubcoreMesh(core_axis_name="c", subcore_axis_name="s")

    @functools.partial(
        pl.kernel,
        out_type=jax.ShapeDtypeStruct((B, D), jnp.float32),
        mesh=mesh,
        scratch_types=[
            pltpu.VMEM((b_per_w,), jnp.int32),
            pltpu.VMEM((C, D), jnp.float32),
            pltpu.VMEM((C, D), jnp.float32),
            pltpu.SemaphoreType.DMA,
            pltpu.SemaphoreType.DMA,
        ],
        compiler_params=pltpu.CompilerParams(use_tc_tiling_on_sc=False),
    )
    def emb(idx_hbm, w_hbm, out_hbm, idx_v, rows0, rows1, sem0, sem1):
        wid = lax.axis_index("s") * nc + lax.axis_index("c")
        base = wid * b_per_w
        pltpu.sync_copy(idx_hbm.at[pl.ds(base, b_per_w)], idx_v)

        pltpu.async_copy(w_hbm.at[idx_v.at[pl.ds(0, C)]], rows0, sem0)

        def body(i, _):
            off = pl.multiple_of(i * C, C)
            nxt = pl.multiple_of((i + 1) * C, C)

            @pl.when(i % 2 == 0)
            def _even():
                @pl.when(i + 1 < nchunks)
                def _():
                    pltpu.async_copy(w_hbm.at[idx_v.at[pl.ds(nxt, C)]],
                                     rows1, sem1)
                pltpu.make_async_copy(w_hbm.at[idx_v.at[pl.ds(off, C)]],
                                      rows0, sem0).wait()
                pltpu.sync_copy(rows0, out_hbm.at[pl.ds(base + off, C)])

            @pl.when(i % 2 == 1)
            def _odd():
                @pl.when(i + 1 < nchunks)
                def _():
                    pltpu.async_copy(w_hbm.at[idx_v.at[pl.ds(nxt, C)]],
                                     rows0, sem0)
                pltpu.make_async_copy(w_hbm.at[idx_v.at[pl.ds(off, C)]],
                                      rows1, sem1).wait()
                pltpu.sync_copy(rows1, out_hbm.at[pl.ds(base + off, C)])

            return 0

        lax.fori_loop(0, nchunks, body, 0)

    return emb


def kernel(input_tensor, weight):
    R, S = input_tensor.shape
    V, D = weight.shape
    B = R * S
    idx_flat = input_tensor.reshape(B).astype(jnp.int32)
    info = plsc.get_sparse_core_info()
    emb = _make_sc_gather(B, V, D, info.num_cores, info.num_subcores)
    out = emb(idx_flat, weight)
    return out.reshape(R, S, D)
